# baseline (speedup 1.0000x reference)
import jax
import jax.numpy as jnp
from jax import lax
from jax.experimental import pallas as pl
from jax.experimental.pallas import tpu as pltpu
from jax.experimental.pallas import tpu_sc as plsc

N = 10000
D = 128
H = 128
NROW = 10240
TRASH = N
E_PAD = 327680
DCH = 128
CH = 64

_NC, _NS = 2, 16
ROWS_PER_SUB = NROW // _NS
DEG_EPW = E_PAD // (_NC * _NS)
DEG_CHUNKS = DEG_EPW // DCH
MAIN_EPS = E_PAD // _NS
MAIN_CHUNKS = MAIN_EPS // CH
NBUF = 4
LEAD = 2
SEG = 8
NSEG = MAIN_CHUNKS // SEG
DEG_LAG = 8

_f32 = jnp.float32
_MESH = dict(core_axis_name="c", subcore_axis_name="s")


def _deg_body(dstp2, ones_hbm, zeros_hbm, deg_out, didx_v, ones_v, semd, cnt_sh):
    cid = lax.axis_index("c")
    sid = lax.axis_index("s")
    wid = sid * _NC + cid
    my_rows = pl.ds(sid * ROWS_PER_SUB, ROWS_PER_SUB)
    pltpu.sync_copy(zeros_hbm, cnt_sh.at[my_rows])
    pltpu.sync_copy(ones_hbm, ones_v)
    pltpu.sync_copy(dstp2.at[pl.ds(wid * DEG_CHUNKS, DEG_CHUNKS)], didx_v)
    plsc.subcore_barrier()

    def step(j, carry):
        pltpu.async_copy(ones_v, cnt_sh.at[didx_v.at[j]], semd, add=True)

        @pl.when(j >= DEG_LAG)
        def _():
            pltpu.make_async_copy(ones_v, cnt_sh.at[didx_v.at[j - DEG_LAG]],
                                  semd).wait()

        return carry

    lax.fori_loop(0, DEG_CHUNKS, step, 0)
    for j in range(DEG_CHUNKS - DEG_LAG, DEG_CHUNKS):
        pltpu.make_async_copy(ones_v, cnt_sh.at[didx_v.at[j]], semd).wait()
    plsc.subcore_barrier()
    out_off = pl.multiple_of(cid * NROW + sid * ROWS_PER_SUB, 8)
    pltpu.sync_copy(cnt_sh.at[my_rows], deg_out.at[pl.ds(out_off, ROWS_PER_SUB)])


_deg_call = pl.kernel(
    _deg_body,
    out_type=jax.ShapeDtypeStruct((2 * NROW, H), _f32),
    mesh=plsc.VectorSubcoreMesh(**_MESH),
    scratch_types=[
        pltpu.VMEM((DEG_CHUNKS, DCH), jnp.int32),
        pltpu.VMEM((DCH, H), _f32),
        pltpu.SemaphoreType.DMA,
        pltpu.VMEM_SHARED((NROW, H), _f32),
    ],
)


def _scatter_body(tcat, srcp2, dstp2, zeros_hbm, out_cat,
                  sidx2, didx_b, rows,
                  sg0, sg1, sg2, sg3,
                  ss0, ss1, ss2, ss3,
                  sd0, sd1, sd2, sd3,
                  si0, si1, acc_sh):
    semg = (sg0, sg1, sg2, sg3)
    sems = (ss0, ss1, ss2, ss3)
    semd = (sd0, sd1, sd2, sd3)
    semi = (si0, si1)
    cid = lax.axis_index("c")
    sid = lax.axis_index("s")
    my_rows = pl.ds(sid * ROWS_PER_SUB, ROWS_PER_SUB)
    srow = cid * (E_PAD // CH) + sid * MAIN_CHUNKS
    drow = sid * MAIN_CHUNKS
    pltpu.sync_copy(zeros_hbm, acc_sh.at[my_rows])
    pltpu.sync_copy(srcp2.at[pl.ds(srow, SEG)], sidx2.at[0])
    pltpu.sync_copy(srcp2.at[pl.ds(srow + SEG, SEG)], sidx2.at[1])
    plsc.subcore_barrier()

    def g_issue(slab, pos, b):
        pltpu.async_copy(tcat.at[sidx2.at[slab, pos]], rows.at[b], semg[b])

    def d_issue(jq, b):
        pltpu.async_copy(dstp2.at[pl.ds(jq, 1)], didx_b.at[pl.ds(b, 1)], semd[b])

    def wait_g(b):
        pltpu.make_async_copy(tcat.at[sidx2.at[0, 0]], rows.at[b], semg[b]).wait()

    def wait_d(b):
        pltpu.make_async_copy(dstp2.at[pl.ds(0, 1)], didx_b.at[pl.ds(b, 1)],
                              semd[b]).wait()

    def wait_s(b):
        pltpu.make_async_copy(rows.at[b], acc_sh.at[didx_b.at[0]], sems[b]).wait()

    def wait_i(p):
        pltpu.make_async_copy(srcp2.at[pl.ds(srow, SEG)], sidx2.at[p],
                              semi[p]).wait()

    for j in range(LEAD):
        g_issue(0, j, j)
        d_issue(drow + j, j)

    def seg_run(s, p):
        for k in range(SEG):
            j = s * SEG + k
            b = k % NBUF
            if k == 0:
                @pl.when(jnp.logical_and(s >= 1, s + 1 < NSEG))
                def _():
                    pltpu.async_copy(srcp2.at[pl.ds(srow + (s + 1) * SEG, SEG)],
                                     sidx2.at[1 - p], semi[1 - p])
            wait_g(b)
            wait_d(b)
            pltpu.async_copy(rows.at[b], acc_sh.at[didx_b.at[b]],
                             sems[b], add=True)

            @pl.when(j >= NBUF - LEAD)
            def _():
                wait_s((k - (NBUF - LEAD)) % NBUF)

            if k == SEG - LEAD - 1:
                @pl.when(jnp.logical_and(s >= 1, s + 1 < NSEG))
                def _():
                    wait_i(1 - p)

            bq = (k + LEAD) % NBUF

            @pl.when(j + LEAD < MAIN_CHUNKS)
            def _():
                if k < SEG - LEAD:
                    g_issue(p, k + LEAD, bq)
                else:
                    g_issue(1 - p, k + LEAD - SEG, bq)
                d_issue(drow + j + LEAD, bq)

    def round_(s2, carry):
        seg_run(2 * s2, 0)
        seg_run(2 * s2 + 1, 1)
        return carry

    lax.fori_loop(0, NSEG // 2, round_, 0)
    for j in range(MAIN_CHUNKS - (NBUF - LEAD), MAIN_CHUNKS):
        wait_s(j % NBUF)
    plsc.subcore_barrier()
    out_off = pl.multiple_of(cid * NROW + sid * ROWS_PER_SUB, 8)
    pltpu.sync_copy(acc_sh.at[my_rows], out_cat.at[pl.ds(out_off, ROWS_PER_SUB)])


_scatter_call = pl.kernel(
    _scatter_body,
    out_type=jax.ShapeDtypeStruct((2 * NROW, H), _f32),
    mesh=plsc.VectorSubcoreMesh(**_MESH),
    scratch_types=[
        pltpu.VMEM((2, SEG, CH), jnp.int32),
        pltpu.VMEM((NBUF, CH), jnp.int32),
        pltpu.VMEM((NBUF, CH, H), _f32),
    ] + [pltpu.SemaphoreType.DMA] * 14 + [
        pltpu.VMEM_SHARED((NROW, H), _f32),
    ],
)


def _xwt_body(x_ref, w_ref, da_ref, db_ref, t_ref):
    deg = da_ref[:, 0:1] + db_ref[:, 0:1] + 1.0
    xs = x_ref[...] * lax.rsqrt(deg)
    t_ref[...] = jnp.dot(xs, w_ref[...], preferred_element_type=_f32)


_RB = 1280
_NB = NROW // _RB

_xwt_call = pl.pallas_call(
    _xwt_body,
    grid=(2 * _NB,),
    in_specs=[
        pl.BlockSpec((_RB, D), lambda i: (i, 0)),
        pl.BlockSpec((D, H), lambda i: (0, 0)),
        pl.BlockSpec((_RB, H), lambda i: (i % _NB, 0)),
        pl.BlockSpec((_RB, H), lambda i: (_NB + i % _NB, 0)),
    ],
    out_specs=pl.BlockSpec((_RB, H), lambda i: (i, 0)),
    out_shape=jax.ShapeDtypeStruct((2 * NROW, H), _f32),
)


def _final_body(acc_ref, t_ref, deg_ref, b_ref, a_ref, disc_ref, out_ref):
    deg = deg_ref[0:NROW, 0:1] + deg_ref[NROW:2 * NROW, 0:1] + 1.0
    dinv = lax.rsqrt(deg)
    b = b_ref[0]
    a = a_ref[0]
    rows = lax.broadcasted_iota(jnp.int32, (NROW, 1), 0)
    mask = rows < N

    hp = dinv * (acc_ref[0:NROW, :] + t_ref[0:NROW, :]) + b[None, :]
    hn = dinv * (acc_ref[NROW:2 * NROW, :] + t_ref[NROW:2 * NROW, :]) + b[None, :]
    pos = jnp.where(hp > 0, hp, a[None, :] * hp)
    neg = jnp.where(hn > 0, hn, a[None, :] * hn)

    pos_m = jnp.where(mask, pos, 0.0)
    summary = jax.nn.sigmoid(jnp.sum(pos_m, axis=0) / N)
    ws = jnp.sum(disc_ref[...] * summary[None, :], axis=1)

    pos_log = jnp.sum(pos * ws[None, :], axis=1, keepdims=True)
    neg_log = jnp.sum(neg * ws[None, :], axis=1, keepdims=True)

    def softplus(z):
        return jnp.maximum(z, 0.0) + jnp.log1p(jnp.exp(-jnp.abs(z)))

    l1 = jnp.sum(jnp.where(mask, softplus(-pos_log), 0.0)) / N
    l2 = jnp.sum(jnp.where(mask, softplus(neg_log), 0.0)) / N
    out_ref[...] = jnp.broadcast_to(l1 + l2, (1, 1))


_final_call = pl.pallas_call(
    _final_body,
    out_shape=jax.ShapeDtypeStruct((1, 1), _f32),
)


def kernel(x, edge_index, conv_W, conv_b, prelu_a, disc_W):
    n = x.shape[0]
    e = edge_index.shape[1]
    perm = jax.random.permutation(jax.random.key(42), n)

    zrows = jnp.zeros((NROW - n, D), _f32)
    x2 = jnp.concatenate([x, zrows, x[perm], zrows])

    src = edge_index[0]
    dst = edge_index[1]
    srcp = jnp.concatenate([src, jnp.zeros((E_PAD - e,), jnp.int32)])
    srcp2 = jnp.concatenate([srcp, srcp + NROW]).reshape(2 * E_PAD // CH, CH)
    dstp = jnp.concatenate([dst, jnp.full((E_PAD - e,), TRASH, jnp.int32)])
    dstp2 = dstp.reshape(E_PAD // CH, CH)
    dstp2d = dstp.reshape(E_PAD // DCH, DCH)

    ones_w = jnp.ones((DCH, H), _f32)
    zeros128 = jnp.zeros((ROWS_PER_SUB, H), _f32)

    deg_cat = _deg_call(dstp2d, ones_w, zeros128)
    tcat = _xwt_call(x2, conv_W, deg_cat, deg_cat)
    acc_cat = _scatter_call(tcat, srcp2, dstp2, zeros128)
    out = _final_call(acc_cat, tcat, deg_cat,
                      conv_b.reshape(1, H), prelu_a.reshape(1, H), disc_W)
    return out.reshape(())


# speedup vs baseline: 1.1286x; 1.1286x over previous
import jax
import jax.numpy as jnp
from jax import lax
from jax.experimental import pallas as pl
from jax.experimental.pallas import tpu as pltpu
from jax.experimental.pallas import tpu_sc as plsc

N = 10000
D = 128
H = 128
NROW = 10240
TRASH = N
E_PAD = 327680
DCH = 128
CH = 32

_NC, _NS = 2, 16
ROWS_PER_SUB = NROW // _NS
DEG_EPW = E_PAD // (_NC * _NS)
DEG_CHUNKS = DEG_EPW // DCH
MAIN_EPS = E_PAD // _NS
MAIN_CHUNKS = MAIN_EPS // CH
NBUF = 8
LEAD = 4
SEG = 8
NSEG = MAIN_CHUNKS // SEG
DEG_LAG = 8

_f32 = jnp.float32
_MESH = dict(core_axis_name="c", subcore_axis_name="s")


def _deg_body(dstp2, ones_hbm, zeros_hbm, deg_out, didx_v, ones_v, semd, cnt_sh):
    cid = lax.axis_index("c")
    sid = lax.axis_index("s")
    wid = sid * _NC + cid
    my_rows = pl.ds(sid * ROWS_PER_SUB, ROWS_PER_SUB)
    pltpu.sync_copy(zeros_hbm, cnt_sh.at[my_rows])
    pltpu.sync_copy(ones_hbm, ones_v)
    pltpu.sync_copy(dstp2.at[pl.ds(wid * DEG_CHUNKS, DEG_CHUNKS)], didx_v)
    plsc.subcore_barrier()

    def step(j, carry):
        pltpu.async_copy(ones_v, cnt_sh.at[didx_v.at[j]], semd, add=True)

        @pl.when(j >= DEG_LAG)
        def _():
            pltpu.make_async_copy(ones_v, cnt_sh.at[didx_v.at[j - DEG_LAG]],
                                  semd).wait()

        return carry

    lax.fori_loop(0, DEG_CHUNKS, step, 0)
    for j in range(DEG_CHUNKS - DEG_LAG, DEG_CHUNKS):
        pltpu.make_async_copy(ones_v, cnt_sh.at[didx_v.at[j]], semd).wait()
    plsc.subcore_barrier()
    out_off = pl.multiple_of(cid * NROW + sid * ROWS_PER_SUB, 8)
    pltpu.sync_copy(cnt_sh.at[my_rows], deg_out.at[pl.ds(out_off, ROWS_PER_SUB)])


_deg_call = pl.kernel(
    _deg_body,
    out_type=jax.ShapeDtypeStruct((2 * NROW, H), _f32),
    mesh=plsc.VectorSubcoreMesh(**_MESH),
    scratch_types=[
        pltpu.VMEM((DEG_CHUNKS, DCH), jnp.int32),
        pltpu.VMEM((DCH, H), _f32),
        pltpu.SemaphoreType.DMA,
        pltpu.VMEM_SHARED((NROW, H), _f32),
    ],
)


def _scatter_body(tcat, srcp2, dstp2, zeros_hbm, out_cat,
                  sidx2, didx_b, rows,
                  sg0, sg1, sg2, sg3, sg4, sg5, sg6, sg7,
                  ss0, ss1, ss2, ss3, ss4, ss5, ss6, ss7,
                  sd0, sd1, sd2, sd3, sd4, sd5, sd6, sd7,
                  si0, si1, acc_sh):
    semg = (sg0, sg1, sg2, sg3, sg4, sg5, sg6, sg7)
    sems = (ss0, ss1, ss2, ss3, ss4, ss5, ss6, ss7)
    semd = (sd0, sd1, sd2, sd3, sd4, sd5, sd6, sd7)
    semi = (si0, si1)
    cid = lax.axis_index("c")
    sid = lax.axis_index("s")
    my_rows = pl.ds(sid * ROWS_PER_SUB, ROWS_PER_SUB)
    srow = cid * (E_PAD // CH) + sid * MAIN_CHUNKS
    drow = sid * MAIN_CHUNKS
    pltpu.sync_copy(zeros_hbm, acc_sh.at[my_rows])
    pltpu.sync_copy(srcp2.at[pl.ds(srow, SEG)], sidx2.at[0])
    pltpu.sync_copy(srcp2.at[pl.ds(srow + SEG, SEG)], sidx2.at[1])
    plsc.subcore_barrier()

    def g_issue(slab, pos, b):
        pltpu.async_copy(tcat.at[sidx2.at[slab, pos]], rows.at[b], semg[b])

    def d_issue(jq, b):
        pltpu.async_copy(dstp2.at[pl.ds(jq, 1)], didx_b.at[pl.ds(b, 1)], semd[b])

    def wait_g(b):
        pltpu.make_async_copy(tcat.at[sidx2.at[0, 0]], rows.at[b], semg[b]).wait()

    def wait_d(b):
        pltpu.make_async_copy(dstp2.at[pl.ds(0, 1)], didx_b.at[pl.ds(b, 1)],
                              semd[b]).wait()

    def wait_s(b):
        pltpu.make_async_copy(rows.at[b], acc_sh.at[didx_b.at[0]], sems[b]).wait()

    def wait_i(p):
        pltpu.make_async_copy(srcp2.at[pl.ds(srow, SEG)], sidx2.at[p],
                              semi[p]).wait()

    for j in range(LEAD):
        g_issue(0, j, j)
        d_issue(drow + j, j)

    def seg_run(s, p):
        for k in range(SEG):
            j = s * SEG + k
            b = k
            if k == 0:
                @pl.when(jnp.logical_and(s >= 1, s + 1 < NSEG))
                def _():
                    pltpu.async_copy(srcp2.at[pl.ds(srow + (s + 1) * SEG, SEG)],
                                     sidx2.at[1 - p], semi[1 - p])
            wait_g(b)
            wait_d(b)
            pltpu.async_copy(rows.at[b], acc_sh.at[didx_b.at[b]],
                             sems[b], add=True)

            @pl.when(j >= NBUF - LEAD)
            def _():
                wait_s((k - (NBUF - LEAD)) % NBUF)

            if k == SEG - LEAD - 1:
                @pl.when(jnp.logical_and(s >= 1, s + 1 < NSEG))
                def _():
                    wait_i(1 - p)

            bq = (k + LEAD) % NBUF

            @pl.when(j + LEAD < MAIN_CHUNKS)
            def _():
                if k < SEG - LEAD:
                    g_issue(p, k + LEAD, bq)
                else:
                    g_issue(1 - p, k + LEAD - SEG, bq)
                d_issue(drow + j + LEAD, bq)

    def round_(s2, carry):
        seg_run(2 * s2, 0)
        seg_run(2 * s2 + 1, 1)
        return carry

    lax.fori_loop(0, NSEG // 2, round_, 0)
    for j in range(MAIN_CHUNKS - (NBUF - LEAD), MAIN_CHUNKS):
        wait_s(j % NBUF)
    plsc.subcore_barrier()
    out_off = pl.multiple_of(cid * NROW + sid * ROWS_PER_SUB, 8)
    pltpu.sync_copy(acc_sh.at[my_rows], out_cat.at[pl.ds(out_off, ROWS_PER_SUB)])


_scatter_call = pl.kernel(
    _scatter_body,
    out_type=jax.ShapeDtypeStruct((2 * NROW, H), _f32),
    mesh=plsc.VectorSubcoreMesh(**_MESH),
    scratch_types=[
        pltpu.VMEM((2, SEG, CH), jnp.int32),
        pltpu.VMEM((NBUF, CH), jnp.int32),
        pltpu.VMEM((NBUF, CH, H), _f32),
    ] + [pltpu.SemaphoreType.DMA] * 26 + [
        pltpu.VMEM_SHARED((NROW, H), _f32),
    ],
)


def _xwt_body(x_ref, w_ref, da_ref, db_ref, t_ref):
    deg = da_ref[:, 0:1] + db_ref[:, 0:1] + 1.0
    xs = x_ref[...] * lax.rsqrt(deg)
    t_ref[...] = jnp.dot(xs, w_ref[...], preferred_element_type=_f32)


_RB = 1280
_NB = NROW // _RB

_xwt_call = pl.pallas_call(
    _xwt_body,
    grid=(2 * _NB,),
    in_specs=[
        pl.BlockSpec((_RB, D), lambda i: (i, 0)),
        pl.BlockSpec((D, H), lambda i: (0, 0)),
        pl.BlockSpec((_RB, H), lambda i: (i % _NB, 0)),
        pl.BlockSpec((_RB, H), lambda i: (_NB + i % _NB, 0)),
    ],
    out_specs=pl.BlockSpec((_RB, H), lambda i: (i, 0)),
    out_shape=jax.ShapeDtypeStruct((2 * NROW, H), _f32),
)


def _final_body(acc_ref, t_ref, deg_ref, b_ref, a_ref, disc_ref, out_ref):
    deg = deg_ref[0:NROW, 0:1] + deg_ref[NROW:2 * NROW, 0:1] + 1.0
    dinv = lax.rsqrt(deg)
    b = b_ref[0]
    a = a_ref[0]
    rows = lax.broadcasted_iota(jnp.int32, (NROW, 1), 0)
    mask = rows < N

    hp = dinv * (acc_ref[0:NROW, :] + t_ref[0:NROW, :]) + b[None, :]
    hn = dinv * (acc_ref[NROW:2 * NROW, :] + t_ref[NROW:2 * NROW, :]) + b[None, :]
    pos = jnp.where(hp > 0, hp, a[None, :] * hp)
    neg = jnp.where(hn > 0, hn, a[None, :] * hn)

    pos_m = jnp.where(mask, pos, 0.0)
    summary = jax.nn.sigmoid(jnp.sum(pos_m, axis=0) / N)
    ws = jnp.sum(disc_ref[...] * summary[None, :], axis=1)

    pos_log = jnp.sum(pos * ws[None, :], axis=1, keepdims=True)
    neg_log = jnp.sum(neg * ws[None, :], axis=1, keepdims=True)

    def softplus(z):
        return jnp.maximum(z, 0.0) + jnp.log1p(jnp.exp(-jnp.abs(z)))

    l1 = jnp.sum(jnp.where(mask, softplus(-pos_log), 0.0)) / N
    l2 = jnp.sum(jnp.where(mask, softplus(neg_log), 0.0)) / N
    out_ref[...] = jnp.broadcast_to(l1 + l2, (1, 1))


_final_call = pl.pallas_call(
    _final_body,
    out_shape=jax.ShapeDtypeStruct((1, 1), _f32),
)


def kernel(x, edge_index, conv_W, conv_b, prelu_a, disc_W):
    n = x.shape[0]
    e = edge_index.shape[1]
    perm = jax.random.permutation(jax.random.key(42), n)

    zrows = jnp.zeros((NROW - n, D), _f32)
    x2 = jnp.concatenate([x, zrows, x[perm], zrows])

    src = edge_index[0]
    dst = edge_index[1]
    srcp = jnp.concatenate([src, jnp.zeros((E_PAD - e,), jnp.int32)])
    srcp2 = jnp.concatenate([srcp, srcp + NROW]).reshape(2 * E_PAD // CH, CH)
    dstp = jnp.concatenate([dst, jnp.full((E_PAD - e,), TRASH, jnp.int32)])
    dstp2 = dstp.reshape(E_PAD // CH, CH)
    dstp2d = dstp.reshape(E_PAD // DCH, DCH)

    ones_w = jnp.ones((DCH, H), _f32)
    zeros128 = jnp.zeros((ROWS_PER_SUB, H), _f32)

    deg_cat = _deg_call(dstp2d, ones_w, zeros128)
    tcat = _xwt_call(x2, conv_W, deg_cat, deg_cat)
    acc_cat = _scatter_call(tcat, srcp2, dstp2, zeros128)
    out = _final_call(acc_cat, tcat, deg_cat,
                      conv_b.reshape(1, H), prelu_a.reshape(1, H), disc_W)
    return out.reshape(())
